# Initial kernel scaffold; baseline (speedup 1.0000x reference)
import jax
import jax.numpy as jnp
from jax.experimental import pallas as pl


def kernel(x, edge_index, edge_attr, w1, w2, w3, b, u1, u2, b2):
    raise NotImplementedError("write your pallas kernel here")


# speedup vs baseline: 1.5652x; 1.5652x over previous
import functools

import jax
import jax.numpy as jnp
from jax import lax
from jax.experimental import pallas as pl
from jax.experimental.pallas import tpu as pltpu
from jax.experimental.pallas import tpu_sc as plsc

_N = 10000
_E = 320000
_D = 128
_H = 64
_DE = 16

_NC, _NS = 2, 16
_EP = 327680
_RT = _EP // 128
_RPT = _RT // _NS
_CR = 8
_CEH = 256
_NCH = _RPT // _CR
_NSTRIDE = 624
_NWIN = 640
_AGG_ROWS = _N + 8


def _node_linear_body(x_ref, w_ref, b_ref, o_ref):
    r = jnp.dot(x_ref[...], w_ref[...], preferred_element_type=jnp.float32)
    r = r + b_ref[...]
    o_ref[0, :, :] = r[:, :_H]
    o_ref[1, :, :] = r[:, _H:]


def _node_linear(x, w1, b):
    bn = 1000
    return pl.pallas_call(
        _node_linear_body,
        grid=(_N // bn,),
        in_specs=[
            pl.BlockSpec((bn, _D), lambda i: (i, 0)),
            pl.BlockSpec((_D, _D), lambda i: (0, 0)),
            pl.BlockSpec((1, _D), lambda i: (0, 0)),
        ],
        out_specs=pl.BlockSpec((2, bn, _H), lambda i: (0, i, 0)),
        out_shape=jax.ShapeDtypeStruct((2, _N, _H), jnp.float32),
    )(x, w1, b)


def _edge_linear_body(ea_ref, w_ref, o_ref):
    r = jnp.dot(ea_ref[...], w_ref[...], preferred_element_type=jnp.float32)
    o_ref[0, :, :] = r[:, :_H]
    o_ref[1, :, :] = r[:, _H:]


def _edge_linear(edge_attr_p, w2):
    be = 2048
    return pl.pallas_call(
        _edge_linear_body,
        grid=(_EP // be,),
        in_specs=[
            pl.BlockSpec((be, _DE), lambda i: (i, 0)),
            pl.BlockSpec((_DE, _D), lambda i: (0, 0)),
        ],
        out_specs=pl.BlockSpec((2, be, _H), lambda i: (0, i, 0)),
        out_shape=jax.ShapeDtypeStruct((2, _EP, _H), jnp.float32),
    )(edge_attr_p, w2)


def _mid_body(h_ref, a_ref, w_ref, o_ref):
    r = (jnp.dot(a_ref[0, :, :], w_ref[:_H, :],
                 preferred_element_type=jnp.float32)
         + jnp.dot(a_ref[1, :, :], w_ref[_H:, :],
                   preferred_element_type=jnp.float32))
    o_ref[0, :, :] = h_ref[0, :, :] + r[:, :_H]
    o_ref[1, :, :] = h_ref[1, :, :] + r[:, _H:]


def _mid(h2, agg2, w3):
    bn = 1000
    return pl.pallas_call(
        _mid_body,
        grid=(_N // bn,),
        in_specs=[
            pl.BlockSpec((2, bn, _H), lambda i: (0, i, 0)),
            pl.BlockSpec((2, bn, _H), lambda i: (0, i, 0)),
            pl.BlockSpec((_D, _D), lambda i: (0, 0)),
        ],
        out_specs=pl.BlockSpec((2, bn, _H), lambda i: (0, i, 0)),
        out_shape=jax.ShapeDtypeStruct((2, _N, _H), jnp.float32),
    )(h2, agg2, w3)


def _final_body(x_ref, u1_ref, a_ref, u2_ref, b2_ref, o_ref):
    r = jnp.dot(x_ref[...], u1_ref[...], preferred_element_type=jnp.float32)
    r = r + jnp.dot(a_ref[0, :, :], u2_ref[:_H, :],
                    preferred_element_type=jnp.float32)
    r = r + jnp.dot(a_ref[1, :, :], u2_ref[_H:, :],
                    preferred_element_type=jnp.float32)
    o_ref[...] = jnp.maximum(r + b2_ref[...], 0.0)


def _final(x, u1, agg2, u2, b2):
    bn = 1000
    return pl.pallas_call(
        _final_body,
        grid=(_N // bn,),
        in_specs=[
            pl.BlockSpec((bn, _D), lambda i: (i, 0)),
            pl.BlockSpec((_D, _D), lambda i: (0, 0)),
            pl.BlockSpec((2, bn, _H), lambda i: (0, i, 0)),
            pl.BlockSpec((_D, _D), lambda i: (0, 0)),
            pl.BlockSpec((1, _D), lambda i: (0, 0)),
        ],
        out_specs=pl.BlockSpec((bn, _D), lambda i: (i, 0)),
        out_shape=jax.ShapeDtypeStruct((_N, _D), jnp.float32),
    )(x, u1, agg2, u2, b2)


def _edge_pass_body(h_hbm, ea_hbm, src_hbm, dst_hbm, out_hbm,
                    agg_s,
                    src_v0, dst_v0, src_v1, dst_v1, ea_v0, ea_v1,
                    rows_v, ld_sem0, ld_sem1, ea_sem0, ea_sem1,
                    g_sem, sc_sem):
    c = lax.axis_index("c")
    s = lax.axis_index("s")
    nb = s * _NSTRIDE
    hoff = c * _N

    @pl.loop(0, _CEH)
    def _zero(i):
        for j in range(_H // 16):
            rows_v[i, pl.ds(j * 16, 16)] = jnp.zeros((16,), jnp.float32)

    for z in range(_NWIN // _CEH):
        pltpu.sync_copy(rows_v.at[pl.ds(0, _CEH)],
                        agg_s.at[pl.ds(nb + z * _CEH, _CEH)])
    pltpu.sync_copy(rows_v.at[pl.ds(0, _NWIN - 2 * _CEH)],
                    agg_s.at[pl.ds(nb + 2 * _CEH, _NWIN - 2 * _CEH)])
    plsc.subcore_barrier()

    rb = s * _RPT
    idx_bufs = ((src_v0, dst_v0, ld_sem0), (src_v1, dst_v1, ld_sem1))
    ea_bufs = ((ea_v0, ea_sem0), (ea_v1, ea_sem1))

    def _idx_loads(ch, srcv, dstv, sem):
        ib = rb + ch * _CR
        return (
            pltpu.make_async_copy(src_hbm.at[pl.ds(ib, _CR)], srcv, sem),
            pltpu.make_async_copy(dst_hbm.at[pl.ds(ib, _CR)], dstv, sem),
        )

    def _ea_load(ch, w, eav, sem):
        eb = (rb + ch * _CR) * 128 + w * _CEH
        return pltpu.make_async_copy(ea_hbm.at[c, pl.ds(eb, _CEH)], eav, sem)

    for k in range(2):
        for cp in _idx_loads(k, *idx_bufs[k]):
            cp.start()
    for w in range(2):
        _ea_load(0, w, *ea_bufs[w]).start()

    @pl.loop(0, _NCH, step=2)
    def _chunks(ch0):
        for k in range(2):
            srcv, dstv, sem = idx_bufs[k]
            ch = ch0 + k
            for cp in _idx_loads(ch, srcv, dstv, sem):
                cp.wait()

            @pl.loop(0, _CR)
            def _rebase(r):
                for j in range(128 // 16):
                    sl = pl.ds(j * 16, 16)
                    srcv[r, sl] = srcv[r, sl] + hoff

            for w in range(4):
                eav, easem = ea_bufs[w % 2]
                _ea_load(ch, w, eav, easem).wait()
                for j in range(2):
                    pltpu.async_copy(h_hbm.at[srcv.at[2 * w + j]],
                                     rows_v.at[pl.ds(j * 128, 128)], g_sem)
                for j in range(2):
                    pltpu.make_async_copy(h_hbm.at[srcv.at[2 * w + j]],
                                          rows_v.at[pl.ds(j * 128, 128)],
                                          g_sem).wait()

                @pl.loop(0, _CEH, unroll=4)
                def _compute(i):
                    for j in range(_H // 16):
                        sl = pl.ds(j * 16, 16)
                        rows_v[i, sl] = jnp.maximum(
                            rows_v[i, sl] + eav[i, sl], 0.0)

                if w < 2:
                    _ea_load(ch, w + 2, eav, easem).start()
                else:
                    @pl.when(ch + 1 < _NCH)
                    def _refill_ea():
                        _ea_load(ch + 1, w - 2, eav, easem).start()

                for j in range(2):
                    pltpu.async_copy(rows_v.at[pl.ds(j * 128, 128)],
                                     agg_s.at[dstv.at[2 * w + j]], sc_sem,
                                     add=True)
                for j in range(2):
                    pltpu.make_async_copy(rows_v.at[pl.ds(j * 128, 128)],
                                          agg_s.at[dstv.at[2 * w + j]],
                                          sc_sem).wait()

            @pl.when(ch + 2 < _NCH)
            def _refill_idx():
                for cp in _idx_loads(ch + 2, srcv, dstv, sem):
                    cp.start()

    plsc.subcore_barrier()
    pltpu.sync_copy(agg_s.at[pl.ds(nb, _NWIN)], out_hbm.at[c, pl.ds(nb, _NWIN)])


@functools.partial(jax.jit, static_argnums=())
def _edge_pass(h2, ea2, src_r, dst_r):
    mesh = plsc.VectorSubcoreMesh(core_axis_name="c", subcore_axis_name="s")
    return pl.kernel(
        _edge_pass_body,
        out_type=jax.ShapeDtypeStruct((2, _N, _H), jnp.float32),
        mesh=mesh,
        compiler_params=pltpu.CompilerParams(use_tc_tiling_on_sc=False),
        scratch_types=[
            pltpu.VMEM_SHARED((_AGG_ROWS, _H), jnp.float32),
            pltpu.VMEM((_CR, 128), jnp.int32),
            pltpu.VMEM((_CR, 128), jnp.int32),
            pltpu.VMEM((_CR, 128), jnp.int32),
            pltpu.VMEM((_CR, 128), jnp.int32),
            pltpu.VMEM((_CEH, _H), jnp.float32),
            pltpu.VMEM((_CEH, _H), jnp.float32),
            pltpu.VMEM((_CEH, _H), jnp.float32),
            pltpu.SemaphoreType.DMA,
            pltpu.SemaphoreType.DMA,
            pltpu.SemaphoreType.DMA,
            pltpu.SemaphoreType.DMA,
            pltpu.SemaphoreType.DMA,
            pltpu.SemaphoreType.DMA,
        ],
    )(h2.reshape(_NC * _N, _H), ea2, src_r, dst_r)


def kernel(x, edge_index, edge_attr, w1, w2, w3, b, u1, u2, b2):
    src = edge_index[0]
    dst = edge_index[1]
    pad = _EP - _E
    src_r = jnp.concatenate(
        [src, jnp.zeros((pad,), jnp.int32)]).reshape(_RT, 128)
    dst_r = jnp.concatenate(
        [dst, jnp.full((pad,), _N, jnp.int32)]).reshape(_RT, 128)
    ea_p = jnp.concatenate(
        [edge_attr, jnp.zeros((pad, _DE), jnp.float32)], axis=0)

    h2 = _node_linear(x, w1, b)
    ea2 = _edge_linear(ea_p, w2)

    def _iter(_, carry):
        h_cur, _agg = carry
        agg = _edge_pass(h_cur, ea2, src_r, dst_r)
        return _mid(h2, agg, w3), agg

    init = (h2, jnp.zeros((2, _N, _H), jnp.float32))
    _, agg2 = lax.fori_loop(0, 2, _iter, init)
    return _final(x, u1, agg2, u2, b2)


# speedup vs baseline: 1.9737x; 1.2610x over previous
import functools

import jax
import jax.numpy as jnp
from jax import lax
from jax.experimental import pallas as pl
from jax.experimental.pallas import tpu as pltpu
from jax.experimental.pallas import tpu_sc as plsc

_N = 10000
_E = 320000
_D = 128
_H = 64
_DE = 16

_NC, _NS = 2, 16
_EP = 327680
_RT = _EP // 128
_RPT = _RT // _NS
_CR = 8
_CEH = 256
_NCH = _RPT // _CR
_NSTRIDE = 624
_NWIN = 640
_AGG_ROWS = _N + 8


def _node_linear_body(x_ref, w_ref, b_ref, o_ref):
    r = jnp.dot(x_ref[...], w_ref[...], preferred_element_type=jnp.float32)
    r = r + b_ref[...]
    o_ref[0, :, :] = r[:, :_H]
    o_ref[1, :, :] = r[:, _H:]


def _node_linear(x, w1, b):
    bn = 1000
    return pl.pallas_call(
        _node_linear_body,
        grid=(_N // bn,),
        in_specs=[
            pl.BlockSpec((bn, _D), lambda i: (i, 0)),
            pl.BlockSpec((_D, _D), lambda i: (0, 0)),
            pl.BlockSpec((1, _D), lambda i: (0, 0)),
        ],
        out_specs=pl.BlockSpec((2, bn, _H), lambda i: (0, i, 0)),
        out_shape=jax.ShapeDtypeStruct((2, _N, _H), jnp.float32),
    )(x, w1, b)


def _edge_linear_body(ea_ref, w_ref, o_ref):
    r = jnp.dot(ea_ref[...], w_ref[...], preferred_element_type=jnp.float32)
    o_ref[0, :, :] = r[:, :_H]
    o_ref[1, :, :] = r[:, _H:]


def _edge_linear(edge_attr_p, w2):
    be = 2048
    return pl.pallas_call(
        _edge_linear_body,
        grid=(_EP // be,),
        in_specs=[
            pl.BlockSpec((be, _DE), lambda i: (i, 0)),
            pl.BlockSpec((_DE, _D), lambda i: (0, 0)),
        ],
        out_specs=pl.BlockSpec((2, be, _H), lambda i: (0, i, 0)),
        out_shape=jax.ShapeDtypeStruct((2, _EP, _H), jnp.float32),
    )(edge_attr_p, w2)


def _mid_body(h_ref, a_ref, w_ref, o_ref):
    r = (jnp.dot(a_ref[0, :, :], w_ref[:_H, :],
                 preferred_element_type=jnp.float32)
         + jnp.dot(a_ref[1, :, :], w_ref[_H:, :],
                   preferred_element_type=jnp.float32))
    o_ref[0, :, :] = h_ref[0, :, :] + r[:, :_H]
    o_ref[1, :, :] = h_ref[1, :, :] + r[:, _H:]


def _mid(h2, agg2, w3):
    bn = 1000
    return pl.pallas_call(
        _mid_body,
        grid=(_N // bn,),
        in_specs=[
            pl.BlockSpec((2, bn, _H), lambda i: (0, i, 0)),
            pl.BlockSpec((2, bn, _H), lambda i: (0, i, 0)),
            pl.BlockSpec((_D, _D), lambda i: (0, 0)),
        ],
        out_specs=pl.BlockSpec((2, bn, _H), lambda i: (0, i, 0)),
        out_shape=jax.ShapeDtypeStruct((2, _N, _H), jnp.float32),
    )(h2, agg2, w3)


def _final_body(x_ref, u1_ref, a_ref, u2_ref, b2_ref, o_ref):
    r = jnp.dot(x_ref[...], u1_ref[...], preferred_element_type=jnp.float32)
    r = r + jnp.dot(a_ref[0, :, :], u2_ref[:_H, :],
                    preferred_element_type=jnp.float32)
    r = r + jnp.dot(a_ref[1, :, :], u2_ref[_H:, :],
                    preferred_element_type=jnp.float32)
    o_ref[...] = jnp.maximum(r + b2_ref[...], 0.0)


def _final(x, u1, agg2, u2, b2):
    bn = 1000
    return pl.pallas_call(
        _final_body,
        grid=(_N // bn,),
        in_specs=[
            pl.BlockSpec((bn, _D), lambda i: (i, 0)),
            pl.BlockSpec((_D, _D), lambda i: (0, 0)),
            pl.BlockSpec((2, bn, _H), lambda i: (0, i, 0)),
            pl.BlockSpec((_D, _D), lambda i: (0, 0)),
            pl.BlockSpec((1, _D), lambda i: (0, 0)),
        ],
        out_specs=pl.BlockSpec((bn, _D), lambda i: (i, 0)),
        out_shape=jax.ShapeDtypeStruct((_N, _D), jnp.float32),
    )(x, u1, agg2, u2, b2)


def _edge_pass_body(h_hbm, ea_hbm, src_hbm, dst_hbm, out_hbm,
                    agg_s,
                    src_v0, dst_v0, src_v1, dst_v1, ea_v0, ea_v1,
                    rows_v0, rows_v1,
                    ld_sem0, ld_sem1, ea_sem0, ea_sem1,
                    g_sem0, g_sem1, sc_sem0, sc_sem1):
    c = lax.axis_index("c")
    s = lax.axis_index("s")
    nb = s * _NSTRIDE
    hoff = c * _N

    @pl.loop(0, _CEH)
    def _zero(i):
        for j in range(_H // 16):
            rows_v0[i, pl.ds(j * 16, 16)] = jnp.zeros((16,), jnp.float32)

    for z in range(_NWIN // _CEH):
        pltpu.sync_copy(rows_v0.at[pl.ds(0, _CEH)],
                        agg_s.at[pl.ds(nb + z * _CEH, _CEH)])
    pltpu.sync_copy(rows_v0.at[pl.ds(0, _NWIN - 2 * _CEH)],
                    agg_s.at[pl.ds(nb + 2 * _CEH, _NWIN - 2 * _CEH)])
    plsc.subcore_barrier()

    rb = s * _RPT
    idx_bufs = ((src_v0, dst_v0, ld_sem0), (src_v1, dst_v1, ld_sem1))
    ea_bufs = ((ea_v0, ea_sem0), (ea_v1, ea_sem1))
    row_bufs = ((rows_v0, g_sem0, sc_sem0), (rows_v1, g_sem1, sc_sem1))

    def _idx_loads(ch, srcv, dstv, sem):
        ib = rb + ch * _CR
        return (
            pltpu.make_async_copy(src_hbm.at[pl.ds(ib, _CR)], srcv, sem),
            pltpu.make_async_copy(dst_hbm.at[pl.ds(ib, _CR)], dstv, sem),
        )

    def _rebase(srcv):
        @pl.loop(0, _CR)
        def _rb_loop(r):
            for j in range(128 // 16):
                sl = pl.ds(j * 16, 16)
                srcv[r, sl] = srcv[r, sl] + hoff

    def _ea_load(ch, w, eav, sem):
        eb = (rb + ch * _CR) * 128 + w * _CEH
        return pltpu.make_async_copy(ea_hbm.at[c, pl.ds(eb, _CEH)], eav, sem)

    def _gathers(srcv, w, rowsv, gsem):
        return tuple(
            pltpu.make_async_copy(h_hbm.at[srcv.at[2 * w + j]],
                                  rowsv.at[pl.ds(j * 128, 128)], gsem)
            for j in range(2))

    def _scatters(dstv, w, rowsv, ssem):
        return tuple(
            pltpu.make_async_copy(rowsv.at[pl.ds(j * 128, 128)],
                                  agg_s.at[dstv.at[2 * w + j]], ssem)
            for j in range(2))

    def _fire_scatters(dstv, w, rowsv, ssem):
        for j in range(2):
            pltpu.async_copy(rowsv.at[pl.ds(j * 128, 128)],
                             agg_s.at[dstv.at[2 * w + j]], ssem, add=True)

    for cp in _idx_loads(0, *idx_bufs[0]):
        cp.start()
    for cp in _idx_loads(0, *idx_bufs[0]):
        cp.wait()
    _rebase(src_v0)
    for cp in _idx_loads(1, *idx_bufs[1]):
        cp.start()
    for w in range(2):
        _ea_load(0, w, *ea_bufs[w]).start()
    for cp in _gathers(src_v0, 0, rows_v0, g_sem0):
        cp.start()

    @pl.loop(0, _NCH, step=2)
    def _chunks(ch0):
        for k in range(2):
            srcv, dstv, sem = idx_bufs[k]
            srcv_n, dstv_n, sem_n = idx_bufs[(k + 1) % 2]
            ch = ch0 + k

            for w in range(4):
                a = w % 2
                b = (w + 1) % 2
                eav, easem = ea_bufs[a]
                rowsv, gsem, ssem = row_bufs[a]
                rowsv_b, gsem_b, ssem_b = row_bufs[b]

                for cp in _gathers(srcv, w, rowsv, gsem):
                    cp.wait()
                _ea_load(ch, w, eav, easem).wait()

                if w == 0:
                    @pl.when(ch > 0)
                    def _drain_prev():
                        for cp in _scatters(dstv_n, 3, rowsv_b, ssem_b):
                            cp.wait()
                        @pl.when(ch + 1 < _NCH)
                        def _idx_fire():
                            for cp in _idx_loads(ch + 1, srcv_n, dstv_n,
                                                 sem_n):
                                cp.start()
                else:
                    for cp in _scatters(dstv, w - 1, rowsv_b, ssem_b):
                        cp.wait()

                if w < 3:
                    for cp in _gathers(srcv, w + 1, rowsv_b, gsem_b):
                        cp.start()
                else:
                    @pl.when(ch + 1 < _NCH)
                    def _gather_next():
                        for cp in _gathers(srcv_n, 0, rowsv_b, gsem_b):
                            cp.start()

                @pl.loop(0, _CEH, unroll=8)
                def _compute(i):
                    for j in range(_H // 16):
                        sl = pl.ds(j * 16, 16)
                        rowsv[i, sl] = jnp.maximum(
                            rowsv[i, sl] + eav[i, sl], 0.0)

                _fire_scatters(dstv, w, rowsv, ssem)
                if w < 2:
                    _ea_load(ch, w + 2, eav, easem).start()
                else:
                    @pl.when(ch + 1 < _NCH)
                    def _refill_ea():
                        _ea_load(ch + 1, w - 2, eav, easem).start()

                if w == 2:
                    @pl.when(ch + 1 < _NCH)
                    def _idx_next():
                        for cp in _idx_loads(ch + 1, srcv_n, dstv_n, sem_n):
                            cp.wait()
                        _rebase(srcv_n)

    for cp in _scatters(idx_bufs[(_NCH - 1) % 2][1], 3, row_bufs[1][0],
                        row_bufs[1][2]):
        cp.wait()

    plsc.subcore_barrier()
    pltpu.sync_copy(agg_s.at[pl.ds(nb, _NWIN)], out_hbm.at[c, pl.ds(nb, _NWIN)])


@functools.partial(jax.jit, static_argnums=())
def _edge_pass(h2, ea2, src_r, dst_r):
    mesh = plsc.VectorSubcoreMesh(core_axis_name="c", subcore_axis_name="s")
    return pl.kernel(
        _edge_pass_body,
        out_type=jax.ShapeDtypeStruct((2, _N, _H), jnp.float32),
        mesh=mesh,
        compiler_params=pltpu.CompilerParams(use_tc_tiling_on_sc=False),
        scratch_types=[
            pltpu.VMEM_SHARED((_AGG_ROWS, _H), jnp.float32),
            pltpu.VMEM((_CR, 128), jnp.int32),
            pltpu.VMEM((_CR, 128), jnp.int32),
            pltpu.VMEM((_CR, 128), jnp.int32),
            pltpu.VMEM((_CR, 128), jnp.int32),
            pltpu.VMEM((_CEH, _H), jnp.float32),
            pltpu.VMEM((_CEH, _H), jnp.float32),
            pltpu.VMEM((_CEH, _H), jnp.float32),
            pltpu.VMEM((_CEH, _H), jnp.float32),
            pltpu.SemaphoreType.DMA,
            pltpu.SemaphoreType.DMA,
            pltpu.SemaphoreType.DMA,
            pltpu.SemaphoreType.DMA,
            pltpu.SemaphoreType.DMA,
            pltpu.SemaphoreType.DMA,
            pltpu.SemaphoreType.DMA,
            pltpu.SemaphoreType.DMA,
        ],
    )(h2.reshape(_NC * _N, _H), ea2, src_r, dst_r)


def kernel(x, edge_index, edge_attr, w1, w2, w3, b, u1, u2, b2):
    src = edge_index[0]
    dst = edge_index[1]
    pad = _EP - _E
    src_r = jnp.concatenate(
        [src, jnp.zeros((pad,), jnp.int32)]).reshape(_RT, 128)
    dst_r = jnp.concatenate(
        [dst, jnp.full((pad,), _N, jnp.int32)]).reshape(_RT, 128)
    ea_p = jnp.concatenate(
        [edge_attr, jnp.zeros((pad, _DE), jnp.float32)], axis=0)

    h2 = _node_linear(x, w1, b)
    ea2 = _edge_linear(ea_p, w2)

    def _iter(_, carry):
        h_cur, _agg = carry
        agg = _edge_pass(h_cur, ea2, src_r, dst_r)
        return _mid(h2, agg, w3), agg

    init = (h2, jnp.zeros((2, _N, _H), jnp.float32))
    _, agg2 = lax.fori_loop(0, 2, _iter, init)
    return _final(x, u1, agg2, u2, b2)


# speedup vs baseline: 2.0941x; 1.0610x over previous
import functools

import jax
import jax.numpy as jnp
from jax import lax
from jax.experimental import pallas as pl
from jax.experimental.pallas import tpu as pltpu
from jax.experimental.pallas import tpu_sc as plsc

_N = 10000
_E = 320000
_D = 128
_H = 64
_DE = 16

_NC, _NS = 2, 16
_EP = 327680
_RT = _EP // 128
_RPT = _RT // _NS
_CR = 8
_CEH = 128
_NCH = _RPT // _CR
_NSTRIDE = 624
_NWIN = 640
_AGG_ROWS = _N + 8


def _node_linear_body(x_ref, w_ref, b_ref, o_ref):
    r = jnp.dot(x_ref[...], w_ref[...], preferred_element_type=jnp.float32)
    r = r + b_ref[...]
    o_ref[0, :, :] = r[:, :_H]
    o_ref[1, :, :] = r[:, _H:]


def _node_linear(x, w1, b):
    bn = 1000
    return pl.pallas_call(
        _node_linear_body,
        grid=(_N // bn,),
        in_specs=[
            pl.BlockSpec((bn, _D), lambda i: (i, 0)),
            pl.BlockSpec((_D, _D), lambda i: (0, 0)),
            pl.BlockSpec((1, _D), lambda i: (0, 0)),
        ],
        out_specs=pl.BlockSpec((2, bn, _H), lambda i: (0, i, 0)),
        out_shape=jax.ShapeDtypeStruct((2, _N, _H), jnp.float32),
    )(x, w1, b)


def _edge_linear_body(ea_ref, w_ref, o_ref):
    r = jnp.dot(ea_ref[...], w_ref[...], preferred_element_type=jnp.float32)
    o_ref[0, :, :] = r[:, :_H]
    o_ref[1, :, :] = r[:, _H:]


def _edge_linear(edge_attr_p, w2):
    be = 2048
    return pl.pallas_call(
        _edge_linear_body,
        grid=(_EP // be,),
        in_specs=[
            pl.BlockSpec((be, _DE), lambda i: (i, 0)),
            pl.BlockSpec((_DE, _D), lambda i: (0, 0)),
        ],
        out_specs=pl.BlockSpec((2, be, _H), lambda i: (0, i, 0)),
        out_shape=jax.ShapeDtypeStruct((2, _EP, _H), jnp.float32),
    )(edge_attr_p, w2)


def _mid_body(h_ref, a_ref, w_ref, o_ref):
    r = (jnp.dot(a_ref[0, :, :], w_ref[:_H, :],
                 preferred_element_type=jnp.float32)
         + jnp.dot(a_ref[1, :, :], w_ref[_H:, :],
                   preferred_element_type=jnp.float32))
    o_ref[0, :, :] = h_ref[0, :, :] + r[:, :_H]
    o_ref[1, :, :] = h_ref[1, :, :] + r[:, _H:]


def _mid(h2, agg2, w3):
    bn = 1000
    return pl.pallas_call(
        _mid_body,
        grid=(_N // bn,),
        in_specs=[
            pl.BlockSpec((2, bn, _H), lambda i: (0, i, 0)),
            pl.BlockSpec((2, bn, _H), lambda i: (0, i, 0)),
            pl.BlockSpec((_D, _D), lambda i: (0, 0)),
        ],
        out_specs=pl.BlockSpec((2, bn, _H), lambda i: (0, i, 0)),
        out_shape=jax.ShapeDtypeStruct((2, _N, _H), jnp.float32),
    )(h2, agg2, w3)


def _final_body(x_ref, u1_ref, a_ref, u2_ref, b2_ref, o_ref):
    r = jnp.dot(x_ref[...], u1_ref[...], preferred_element_type=jnp.float32)
    r = r + jnp.dot(a_ref[0, :, :], u2_ref[:_H, :],
                    preferred_element_type=jnp.float32)
    r = r + jnp.dot(a_ref[1, :, :], u2_ref[_H:, :],
                    preferred_element_type=jnp.float32)
    o_ref[...] = jnp.maximum(r + b2_ref[...], 0.0)


def _final(x, u1, agg2, u2, b2):
    bn = 1000
    return pl.pallas_call(
        _final_body,
        grid=(_N // bn,),
        in_specs=[
            pl.BlockSpec((bn, _D), lambda i: (i, 0)),
            pl.BlockSpec((_D, _D), lambda i: (0, 0)),
            pl.BlockSpec((2, bn, _H), lambda i: (0, i, 0)),
            pl.BlockSpec((_D, _D), lambda i: (0, 0)),
            pl.BlockSpec((1, _D), lambda i: (0, 0)),
        ],
        out_specs=pl.BlockSpec((bn, _D), lambda i: (i, 0)),
        out_shape=jax.ShapeDtypeStruct((_N, _D), jnp.float32),
    )(x, u1, agg2, u2, b2)


_NQ = (_CR * 128) // _CEH


def _edge_pass_body(h_hbm, ea_hbm, src_hbm, dst_hbm, out_hbm,
                    h_s, agg_s,
                    src_v0, dst_v0, src_v1, dst_v1, ea_v0, ea_v1,
                    rows_v0, rows_v1,
                    ld_sem0, ld_sem1, ea_sem0, ea_sem1,
                    g_sem0, g_sem1, sc_sem0, sc_sem1):
    c = lax.axis_index("c")
    s = lax.axis_index("s")
    nb = s * _NSTRIDE

    pltpu.sync_copy(h_hbm.at[c, pl.ds(nb, _NWIN)], h_s.at[pl.ds(nb, _NWIN)])

    @pl.loop(0, _CEH)
    def _zero(i):
        for j in range(_H // 16):
            rows_v0[i, pl.ds(j * 16, 16)] = jnp.zeros((16,), jnp.float32)

    for z in range(_NWIN // _CEH):
        pltpu.sync_copy(rows_v0.at[pl.ds(0, _CEH)],
                        agg_s.at[pl.ds(nb + z * _CEH, _CEH)])
    plsc.subcore_barrier()

    rb = s * _RPT
    idx_bufs = ((src_v0, dst_v0, ld_sem0), (src_v1, dst_v1, ld_sem1))
    ea_bufs = ((ea_v0, ea_sem0), (ea_v1, ea_sem1))
    row_bufs = ((rows_v0, g_sem0, sc_sem0), (rows_v1, g_sem1, sc_sem1))

    def _idx_loads(ch, srcv, dstv, sem):
        ib = rb + ch * _CR
        return (
            pltpu.make_async_copy(src_hbm.at[pl.ds(ib, _CR)], srcv, sem),
            pltpu.make_async_copy(dst_hbm.at[pl.ds(ib, _CR)], dstv, sem),
        )

    def _ea_load(ch, w, eav, sem):
        eb = (rb + ch * _CR) * 128 + w * _CEH
        return pltpu.make_async_copy(ea_hbm.at[c, pl.ds(eb, _CEH)], eav, sem)

    def _gather(srcv, w, rowsv, gsem):
        return pltpu.make_async_copy(h_s.at[srcv.at[w]], rowsv, gsem)

    def _scatter(dstv, w, rowsv, ssem):
        return pltpu.make_async_copy(rowsv, agg_s.at[dstv.at[w]], ssem)

    for cp in _idx_loads(0, *idx_bufs[0]):
        cp.start()
    for cp in _idx_loads(0, *idx_bufs[0]):
        cp.wait()
    for cp in _idx_loads(1, *idx_bufs[1]):
        cp.start()
    for w in range(2):
        _ea_load(0, w, *ea_bufs[w]).start()
    _gather(src_v0, 0, rows_v0, g_sem0).start()

    @pl.loop(0, _NCH, step=2)
    def _chunks(ch0):
        for k in range(2):
            srcv, dstv, sem = idx_bufs[k]
            srcv_n, dstv_n, sem_n = idx_bufs[(k + 1) % 2]
            ch = ch0 + k

            for w in range(_NQ):
                a = w % 2
                b = (w + 1) % 2
                eav, easem = ea_bufs[a]
                rowsv, gsem, ssem = row_bufs[a]
                rowsv_b, gsem_b, ssem_b = row_bufs[b]

                _gather(srcv, w, rowsv, gsem).wait()
                _ea_load(ch, w, eav, easem).wait()

                if w == 0:
                    @pl.when(ch > 0)
                    def _drain_prev():
                        _scatter(dstv_n, _NQ - 1, rowsv_b, ssem_b).wait()
                        @pl.when(ch + 1 < _NCH)
                        def _idx_fire():
                            for cp in _idx_loads(ch + 1, srcv_n, dstv_n,
                                                 sem_n):
                                cp.start()
                else:
                    _scatter(dstv, w - 1, rowsv_b, ssem_b).wait()

                if w < _NQ - 1:
                    _gather(srcv, w + 1, rowsv_b, gsem_b).start()
                else:
                    @pl.when(ch + 1 < _NCH)
                    def _gather_next():
                        _gather(srcv_n, 0, rowsv_b, gsem_b).start()

                @pl.loop(0, _CEH, unroll=8)
                def _compute(i):
                    for j in range(_H // 16):
                        sl = pl.ds(j * 16, 16)
                        rowsv[i, sl] = jnp.maximum(
                            rowsv[i, sl] + eav[i, sl], 0.0)

                pltpu.async_copy(rowsv, agg_s.at[dstv.at[w]], ssem, add=True)
                if w < _NQ - 2:
                    _ea_load(ch, w + 2, eav, easem).start()
                else:
                    @pl.when(ch + 1 < _NCH)
                    def _refill_ea():
                        _ea_load(ch + 1, w - (_NQ - 2), eav, easem).start()

                if w == _NQ - 2:
                    @pl.when(ch + 1 < _NCH)
                    def _idx_next():
                        for cp in _idx_loads(ch + 1, srcv_n, dstv_n, sem_n):
                            cp.wait()

    for cp in (_scatter(idx_bufs[(_NCH - 1) % 2][1], _NQ - 1,
                        row_bufs[(_NQ - 1) % 2][0],
                        row_bufs[(_NQ - 1) % 2][2]),):
        cp.wait()

    plsc.subcore_barrier()
    pltpu.sync_copy(agg_s.at[pl.ds(nb, _NWIN)], out_hbm.at[c, pl.ds(nb, _NWIN)])


@functools.partial(jax.jit, static_argnums=())
def _edge_pass(h2, ea2, src_r, dst_r):
    mesh = plsc.VectorSubcoreMesh(core_axis_name="c", subcore_axis_name="s")
    return pl.kernel(
        _edge_pass_body,
        out_type=jax.ShapeDtypeStruct((2, _N, _H), jnp.float32),
        mesh=mesh,
        compiler_params=pltpu.CompilerParams(use_tc_tiling_on_sc=False),
        scratch_types=[
            pltpu.VMEM_SHARED((_N, _H), jnp.float32),
            pltpu.VMEM_SHARED((_AGG_ROWS, _H), jnp.float32),
            pltpu.VMEM((_CR, 128), jnp.int32),
            pltpu.VMEM((_CR, 128), jnp.int32),
            pltpu.VMEM((_CR, 128), jnp.int32),
            pltpu.VMEM((_CR, 128), jnp.int32),
            pltpu.VMEM((_CEH, _H), jnp.float32),
            pltpu.VMEM((_CEH, _H), jnp.float32),
            pltpu.VMEM((_CEH, _H), jnp.float32),
            pltpu.VMEM((_CEH, _H), jnp.float32),
            pltpu.SemaphoreType.DMA,
            pltpu.SemaphoreType.DMA,
            pltpu.SemaphoreType.DMA,
            pltpu.SemaphoreType.DMA,
            pltpu.SemaphoreType.DMA,
            pltpu.SemaphoreType.DMA,
            pltpu.SemaphoreType.DMA,
            pltpu.SemaphoreType.DMA,
        ],
    )(h2, ea2, src_r, dst_r)


def kernel(x, edge_index, edge_attr, w1, w2, w3, b, u1, u2, b2):
    src = edge_index[0]
    dst = edge_index[1]
    pad = _EP - _E
    src_r = jnp.concatenate(
        [src, jnp.zeros((pad,), jnp.int32)]).reshape(_RT, 128)
    dst_r = jnp.concatenate(
        [dst, jnp.full((pad,), _N, jnp.int32)]).reshape(_RT, 128)
    ea_p = jnp.concatenate(
        [edge_attr, jnp.zeros((pad, _DE), jnp.float32)], axis=0)

    h2 = _node_linear(x, w1, b)
    ea2 = _edge_linear(ea_p, w2)

    def _iter(_, carry):
        h_cur, _agg = carry
        agg = _edge_pass(h_cur, ea2, src_r, dst_r)
        return _mid(h2, agg, w3), agg

    init = (h2, jnp.zeros((2, _N, _H), jnp.float32))
    _, agg2 = lax.fori_loop(0, 2, _iter, init)
    return _final(x, u1, agg2, u2, b2)
